# Initial kernel scaffold; baseline (speedup 1.0000x reference)
import jax
import jax.numpy as jnp
from jax.experimental import pallas as pl


def kernel(x, w_qs, w_ks):
    raise NotImplementedError("write your pallas kernel here")


# speedup vs baseline: 2.3916x; 2.3916x over previous
import jax
import jax.numpy as jnp
from jax.experimental import pallas as pl
from jax.experimental.pallas import tpu as pltpu

B, T, D = 4, 512, 1024
N, H = 16, 256


def _fused_attn_kernel(x_ref, w_ref, e_ref, a_ref):
    xb = x_ref[0]
    wb = w_ref[0]
    wq = jnp.dot(xb, wb, preferred_element_type=jnp.float32)
    scores = jax.lax.dot_general(
        wq, wq, (((1,), (1,)), ((), ())),
        preferred_element_type=jnp.float32)
    m = jnp.max(scores, axis=-1, keepdims=True)
    e = jnp.exp(scores - m)
    attn = e / jnp.sum(e, axis=-1, keepdims=True)
    a_ref[0, 0] = attn
    e_ref[0, 0] = jnp.dot(attn, xb, preferred_element_type=jnp.float32)


def kernel(x, w_qs, w_ks):
    del w_ks
    e_agg, attn = pl.pallas_call(
        _fused_attn_kernel,
        grid=(B, N),
        in_specs=[
            pl.BlockSpec((1, T, D), lambda b, n: (b, 0, 0)),
            pl.BlockSpec((1, D, H), lambda b, n: (n, 0, 0)),
        ],
        out_specs=[
            pl.BlockSpec((1, 1, T, D), lambda b, n: (b, n, 0, 0)),
            pl.BlockSpec((1, 1, T, T), lambda b, n: (b, n, 0, 0)),
        ],
        out_shape=[
            jax.ShapeDtypeStruct((B, N, T, D), jnp.float32),
            jax.ShapeDtypeStruct((B, N, T, T), jnp.float32),
        ],
        compiler_params=pltpu.CompilerParams(
            dimension_semantics=("parallel", "parallel"),
        ),
    )(x, w_qs)
    return e_agg, attn


# speedup vs baseline: 2.6154x; 1.0936x over previous
import jax
import jax.numpy as jnp
from jax.experimental import pallas as pl
from jax.experimental.pallas import tpu as pltpu

B, T, D = 4, 512, 1024
N, H = 16, 256


def _fused_attn_kernel(x_ref, w_ref, e_ref, a_ref):
    n = pl.program_id(1)
    xb = x_ref[0]
    wb = w_ref[n]
    wq = jnp.dot(xb, wb, preferred_element_type=jnp.float32)
    scores = jax.lax.dot_general(
        wq, wq, (((1,), (1,)), ((), ())),
        preferred_element_type=jnp.float32)
    m = jnp.max(scores, axis=-1, keepdims=True)
    e = jnp.exp(scores - m)
    attn = e / jnp.sum(e, axis=-1, keepdims=True)
    a_ref[0, 0] = attn
    e_ref[0, 0] = jnp.dot(attn, xb, preferred_element_type=jnp.float32)


def kernel(x, w_qs, w_ks):
    del w_ks
    e_agg, attn = pl.pallas_call(
        _fused_attn_kernel,
        grid=(B, N),
        in_specs=[
            pl.BlockSpec((1, T, D), lambda b, n: (b, 0, 0)),
            pl.BlockSpec((N, D, H), lambda b, n: (0, 0, 0)),
        ],
        out_specs=[
            pl.BlockSpec((1, 1, T, D), lambda b, n: (b, n, 0, 0)),
            pl.BlockSpec((1, 1, T, T), lambda b, n: (b, n, 0, 0)),
        ],
        out_shape=[
            jax.ShapeDtypeStruct((B, N, T, D), jnp.float32),
            jax.ShapeDtypeStruct((B, N, T, T), jnp.float32),
        ],
        compiler_params=pltpu.CompilerParams(
            dimension_semantics=("parallel", "parallel"),
        ),
    )(x, w_qs)
    return e_agg, attn
